# baseline (speedup 1.0000x reference)
import jax
import jax.numpy as jnp
from jax.experimental import pallas as pl

BN_EPS = 1e-5
COS_EPS = 1e-8
BLOCK_E = 4096


def _pae_block(x_ref, w1_ref, b1_ref, w2_ref, b2_ref, o_ref):
    x = x_ref[...]
    w1 = w1_ref[...]
    b1 = b1_ref[...]
    w2 = w2_ref[...]
    b2 = b2_ref[...]
    in_dim = w1.shape[0]
    x1 = x[:, :in_dim]
    x2 = x[:, in_dim:]
    a1 = jnp.maximum(jnp.dot(x1, w1, preferred_element_type=jnp.float32) + b1, 0.0)
    a2 = jnp.maximum(jnp.dot(x2, w1, preferred_element_type=jnp.float32) + b1, 0.0)
    h1 = jnp.dot(a1, w2, preferred_element_type=jnp.float32) + b2
    h2 = jnp.dot(a2, w2, preferred_element_type=jnp.float32) + b2
    s11 = jnp.sum(h1 * h1, axis=1)
    s22 = jnp.sum(h2 * h2, axis=1)
    s12 = jnp.sum(h1 * h2, axis=1)
    n1 = jnp.maximum(jnp.sqrt(s11), COS_EPS)
    n2 = jnp.maximum(jnp.sqrt(s22), COS_EPS)
    o_ref[...] = (s12 / (n1 * n2) + 1.0) * 0.5


def kernel(edge_index, edgenet_input, flag, W1, b1, gamma, beta,
           running_mean, running_var, W2, b2):
    n_edges, feat = edgenet_input.shape
    in_dim = feat // 2
    hidden = W1.shape[1]

    scale = gamma * jax.lax.rsqrt(running_var + BN_EPS)
    w2f = W2 * scale[:, None]
    b2f = b2 + (beta - running_mean * scale) @ W2

    edge_weight = pl.pallas_call(
        _pae_block,
        grid=(pl.cdiv(n_edges, BLOCK_E),),
        in_specs=[
            pl.BlockSpec((BLOCK_E, feat), lambda i: (i, 0)),
            pl.BlockSpec((in_dim, hidden), lambda i: (0, 0)),
            pl.BlockSpec((1, hidden), lambda i: (0, 0)),
            pl.BlockSpec((hidden, hidden), lambda i: (0, 0)),
            pl.BlockSpec((1, hidden), lambda i: (0, 0)),
        ],
        out_specs=pl.BlockSpec((BLOCK_E,), lambda i: (i,)),
        out_shape=jax.ShapeDtypeStruct((n_edges,), jnp.float32),
    )(edgenet_input, W1, b1.reshape(1, hidden), w2f, b2f.reshape(1, hidden))

    return edge_weight, edge_index
